# baseline (speedup 1.0000x reference)
import functools

import jax
import jax.numpy as jnp
from jax.experimental import pallas as pl
from jax.experimental.pallas import tpu as pltpu

_MT = 400
_RC = 80


def _hi_dot(x, w):
    return jax.lax.dot_general(
        x, w, (((1,), (0,)), ((), ())),
        precision=jax.lax.Precision.HIGHEST,
        preferred_element_type=jnp.float32)


def _body(x_ref, w1_ref, b1_ref, w2_ref, b2_ref, adj_ref, o_ref,
          s1_ref, h2_ref, *, ni):
    g = pl.program_id(0)
    nc = _MT // _RC

    @pl.when(g == 0)
    def _():
        def chunk(k, _):
            x = x_ref[pl.ds(k * 1000, 1000), :]
            s1_ref[pl.ds(k * 1000, 1000), :] = (
                _hi_dot(x, w1_ref[...]).astype(jnp.bfloat16))
            return 0

        jax.lax.fori_loop(0, x_ref.shape[0] // 1000, chunk, 0)

    @pl.when((g >= 1) & (g <= ni))
    def _():
        def chunk(k, _):
            a = adj_ref[pl.ds(k * _RC, _RC), :].astype(jnp.bfloat16)
            acc = jnp.dot(a, s1_ref[...], preferred_element_type=jnp.float32)
            h = jnp.maximum(acc + b1_ref[...], 0.0)
            h2_ref[pl.ds((g - 1) * _MT + k * _RC, _RC), :] = (
                _hi_dot(h, w2_ref[...]).astype(jnp.bfloat16))
            return 0

        jax.lax.fori_loop(0, nc, chunk, 0)

    @pl.when(g > ni)
    def _():
        def chunk(k, _):
            a = adj_ref[pl.ds(k * _RC, _RC), :].astype(jnp.bfloat16)
            x = jnp.dot(a, h2_ref[...], preferred_element_type=jnp.float32)
            x = x + b2_ref[...]
            m = jnp.max(x, axis=1, keepdims=True)
            s = x - m
            o_ref[pl.ds(k * _RC, _RC), :] = (
                s - jnp.log(jnp.sum(jnp.exp(s), axis=1, keepdims=True)))
            return 0

        jax.lax.fori_loop(0, nc, chunk, 0)


def kernel(feature, adj, W1, b1, W2, b2):
    n, d_in = feature.shape
    d_hid = W1.shape[1]
    d_out = W2.shape[1]
    ni = n // _MT

    one = pl.Buffered(buffer_count=1)

    def adj_idx(g):
        return (jnp.where(g == 0, 0, (g - 1) % ni), 0)

    def out_idx(g):
        return (jnp.where(g <= ni, 0, g - ni - 1), 0)

    return pl.pallas_call(
        functools.partial(_body, ni=ni),
        grid=(2 * ni + 1,),
        in_specs=[
            pl.BlockSpec((n, d_in), lambda g: (0, 0), pipeline_mode=one),
            pl.BlockSpec((d_in, d_hid), lambda g: (0, 0), pipeline_mode=one),
            pl.BlockSpec((1, d_hid), lambda g: (0, 0), pipeline_mode=one),
            pl.BlockSpec((d_hid, d_out), lambda g: (0, 0), pipeline_mode=one),
            pl.BlockSpec((1, d_out), lambda g: (0, 0), pipeline_mode=one),
            pl.BlockSpec((_MT, n), adj_idx),
        ],
        out_specs=pl.BlockSpec((_MT, d_out), out_idx),
        out_shape=jax.ShapeDtypeStruct((n, d_out), jnp.float32),
        scratch_shapes=[
            pltpu.VMEM((n, d_hid), jnp.bfloat16),
            pltpu.VMEM((n, d_out), jnp.bfloat16),
        ],
        compiler_params=pltpu.CompilerParams(
            dimension_semantics=("arbitrary",)),
    )(feature, W1, b1.reshape(1, -1), W2, b2.reshape(1, -1), adj)


# speedup vs baseline: 1.1151x; 1.1151x over previous
import functools

import jax
import jax.numpy as jnp
from jax.experimental import pallas as pl
from jax.experimental.pallas import tpu as pltpu

_MT = 200
_DEPTH = 4


def _hi_dot(x, w):
    return jax.lax.dot_general(
        x, w, (((1,), (0,)), ((), ())),
        precision=jax.lax.Precision.HIGHEST,
        preferred_element_type=jnp.float32)


def _body(x_ref, w1_ref, b1_ref, w2_ref, b2_ref, adj_ref, o_ref,
          buf_ref, sem_ref, s1_ref, h2_ref):
    n = x_ref.shape[0]
    nb = n // _MT

    def copy(slot, idx):
        return pltpu.make_async_copy(
            adj_ref.at[pl.ds((idx % nb) * _MT, _MT), :],
            buf_ref.at[slot],
            sem_ref.at[slot],
        )

    for s in range(_DEPTH):
        copy(s, s).start()

    def s1_chunk(k, _):
        x = x_ref[pl.ds(k * 1000, 1000), :]
        s1_ref[pl.ds(k * 1000, 1000), :] = (
            _hi_dot(x, w1_ref[...]).astype(jnp.bfloat16))
        return 0

    jax.lax.fori_loop(0, n // 1000, s1_chunk, 0)

    def loop(b, _):
        slot = jax.lax.rem(b, _DEPTH)
        copy(slot, b).wait()
        row = jax.lax.rem(b, nb) * _MT

        @pl.when(b < nb)
        def _():
            a = buf_ref[slot].astype(jnp.bfloat16)
            acc = jnp.dot(a, s1_ref[...], preferred_element_type=jnp.float32)
            h = jnp.maximum(acc + b1_ref[...], 0.0)
            h2_ref[pl.ds(row, _MT), :] = (
                _hi_dot(h, w2_ref[...]).astype(jnp.bfloat16))

        @pl.when(b >= nb)
        def _():
            a = buf_ref[slot].astype(jnp.bfloat16)
            x = jnp.dot(a, h2_ref[...], preferred_element_type=jnp.float32)
            x = x + b2_ref[...]
            m = jnp.max(x, axis=1, keepdims=True)
            s = x - m
            o_ref[pl.ds(row, _MT), :] = (
                s - jnp.log(jnp.sum(jnp.exp(s), axis=1, keepdims=True)))

        nxt = b + _DEPTH

        @pl.when(nxt < 2 * nb)
        def _():
            copy(slot, nxt).start()

        return 0

    jax.lax.fori_loop(0, 2 * nb, loop, 0)


def kernel(feature, adj, W1, b1, W2, b2):
    n, d_in = feature.shape
    d_hid = W1.shape[1]
    d_out = W2.shape[1]

    return pl.pallas_call(
        _body,
        in_specs=[
            pl.BlockSpec(memory_space=pltpu.MemorySpace.VMEM),
            pl.BlockSpec(memory_space=pltpu.MemorySpace.VMEM),
            pl.BlockSpec(memory_space=pltpu.MemorySpace.VMEM),
            pl.BlockSpec(memory_space=pltpu.MemorySpace.VMEM),
            pl.BlockSpec(memory_space=pltpu.MemorySpace.VMEM),
            pl.BlockSpec(memory_space=pltpu.MemorySpace.HBM),
        ],
        out_specs=pl.BlockSpec(memory_space=pltpu.MemorySpace.VMEM),
        out_shape=jax.ShapeDtypeStruct((n, d_out), jnp.float32),
        scratch_shapes=[
            pltpu.VMEM((_DEPTH, _MT, n), jnp.float32),
            pltpu.SemaphoreType.DMA((_DEPTH,)),
            pltpu.VMEM((n, d_hid), jnp.bfloat16),
            pltpu.VMEM((n, d_out), jnp.bfloat16),
        ],
    )(feature, W1, b1.reshape(1, -1), W2, b2.reshape(1, -1), adj)
